# baseline (speedup 1.0000x reference)
import math

import jax
import jax.numpy as jnp
import numpy as np
from jax import lax
from jax.experimental import pallas as pl
from jax.experimental.pallas import tpu as pltpu
from jax.experimental.pallas import tpu_sc as plsc

_B = 4
_S = 4096
_D = 1024
_MAX_LEN = 4096
_TS = 512
_NS = _S // _TS
_LOG_FACTOR = -math.log(10000.0) / _D

_S_SC = 512
_NW = 32
_RPW = _S_SC // _NW
_RB = 16
_BLK = _RB * _D


def _dim_rows():
    d = jax.lax.broadcasted_iota(jnp.int32, (1, _D), 1)
    freq = jnp.exp(((d // 2) * 2).astype(jnp.float32) * _LOG_FACTOR)
    phase = (d % 2).astype(jnp.float32) * (math.pi / 2)
    return freq, phase


def _pe_kernel(x_ref, emb_ref, o_ref, v_ref, w_ref):
    s = pl.program_id(0)

    @pl.when(s == 0)
    def _init_pe():
        pos = jax.lax.broadcasted_iota(jnp.int32, (8, 1), 0).astype(jnp.float32)
        freq, phase = _dim_rows()
        angle = pos * freq + phase
        v_ref[0:8, :] = jnp.sin(angle)
        w_ref[0:8, :] = jnp.cos(angle)
        k = 8
        while k < _TS:
            c = jnp.cos(k * freq)
            sn = jnp.sin(k * freq)
            v0 = v_ref[0:k, :]
            w0 = w_ref[0:k, :]
            v_ref[k:2 * k, :] = v0 * c + w0 * sn
            w_ref[k:2 * k, :] = w0 * c - v0 * sn
            k *= 2

    @pl.when(s > 0)
    def _advance_pe():
        freq, _ = _dim_rows()
        c = jnp.cos(_TS * freq)
        sn = jnp.sin(_TS * freq)
        v = v_ref[...]
        w = w_ref[...]
        v_ref[...] = v * c + w * sn
        w_ref[...] = w * c - v * sn

    pe = v_ref[...] + emb_ref[0, :][None, :]
    o_ref[...] = x_ref[...] + pe[None, :, :]


def _tc_kernel(x, node_emb):
    return pl.pallas_call(
        _pe_kernel,
        grid=(_NS,),
        in_specs=[
            pl.BlockSpec((_B, _TS, _D), lambda s: (0, s, 0)),
            pl.BlockSpec((5, _D), lambda s: (0, 0)),
        ],
        out_specs=pl.BlockSpec((_B, _TS, _D), lambda s: (0, s, 0)),
        out_shape=jax.ShapeDtypeStruct((_B, _S, _D), jnp.float32),
        scratch_shapes=[
            pltpu.VMEM((_TS, _D), jnp.float32),
            pltpu.VMEM((_TS, _D), jnp.float32),
        ],
    )(x, node_emb)


def _sinusoid_table():
    position = np.arange(0, _S_SC, dtype=np.float64)[:, None]
    div_term = np.exp(np.arange(0, _D, 2, dtype=np.float64)
                      * (-math.log(10000.0) / _D))
    enc = np.zeros((_S_SC, _D), dtype=np.float32)
    enc[:, 0::2] = np.sin(position * div_term)
    enc[:, 1::2] = np.cos(position * div_term)
    return jnp.asarray(enc.reshape(-1))


def _sc_body(x_hbm, emb_hbm, enc_hbm, out_hbm,
             emb16_v, enc_v, x_v, o_v, sx0, sx1, so0, so1):
    c = lax.axis_index("c")
    s = lax.axis_index("s")
    wid = s * 2 + c
    base = wid * _RPW * _D
    x_sems = (sx0, sx1)
    o_sems = (so0, so1)

    for r in range(_RB):
        pltpu.sync_copy(emb_hbm.at[0], emb16_v.at[pl.ds(r * _D, _D)])
    pltpu.sync_copy(enc_hbm.at[pl.ds(base, _BLK)], enc_v)

    x_descs = [None] * _B
    o_descs = [None] * _B

    def start_x(b):
        x_descs[b] = pltpu.async_copy(
            x_hbm.at[b, pl.ds(base, _BLK)], x_v.at[b % 2], x_sems[b % 2])

    start_x(0)
    start_x(1)
    for b in range(_B):
        slot = b % 2
        x_descs[b].wait()
        if b >= 2:
            o_descs[b - 2].wait()

        @plsc.parallel_loop(0, _BLK, step=16, unroll=8)
        def _(i, slot=slot):
            o_v[slot, pl.ds(i, 16)] = (x_v[slot, pl.ds(i, 16)]
                                       + enc_v[pl.ds(i, 16)]
                                       + emb16_v[pl.ds(i, 16)])

        o_descs[b] = pltpu.async_copy(
            o_v.at[slot], out_hbm.at[b, pl.ds(base, _BLK)], o_sems[slot])
        if b + 2 < _B:
            start_x(b + 2)

    o_descs[-2].wait()
    o_descs[-1].wait()


def _sc_kernel(x2, node_emb, enc):
    return pl.kernel(
        _sc_body,
        out_type=jax.ShapeDtypeStruct((_B, _S_SC * _D), jnp.float32),
        mesh=plsc.VectorSubcoreMesh(core_axis_name="c", subcore_axis_name="s"),
        scratch_types=[
            pltpu.VMEM((_BLK,), jnp.float32),
            pltpu.VMEM((_BLK,), jnp.float32),
            pltpu.VMEM((2, _BLK), jnp.float32),
            pltpu.VMEM((2, _BLK), jnp.float32),
            pltpu.SemaphoreType.DMA,
            pltpu.SemaphoreType.DMA,
            pltpu.SemaphoreType.DMA,
            pltpu.SemaphoreType.DMA,
        ],
    )(x2, node_emb, enc)


def kernel(x, node_emb):
    enc = _sinusoid_table()
    x2 = x.reshape(_B, _S * _D)
    tc_out = _tc_kernel(x, node_emb)
    sc_out = _sc_kernel(x2, node_emb, enc)
    return {"tc": tc_out, "sc": sc_out}
